# baseline (speedup 1.0000x reference)
import functools

import jax
import jax.numpy as jnp
from jax import lax
from jax.experimental import pallas as pl
from jax.experimental.pallas import tpu as pltpu
from jax.experimental.pallas import tpu_sc as plsc

_NC = 2
_NS = 16
_NW = _NC * _NS


def _sc_gather_sum(idx3, nchunks2, emb, L, D, chunk_pos):
    n_chunks = idx3.shape[1]
    pos_per_w = n_chunks * chunk_pos
    rows_per_chunk = chunk_pos * L
    mesh = plsc.VectorSubcoreMesh(core_axis_name="c", subcore_axis_name="s")

    @functools.partial(
        pl.kernel,
        out_type=jax.ShapeDtypeStruct((_NW, pos_per_w, D), jnp.float32),
        mesh=mesh,
        scratch_types=[
            pltpu.VMEM((n_chunks, rows_per_chunk), jnp.int32),
            pltpu.VMEM((16,), jnp.int32),
            pltpu.VMEM((2, rows_per_chunk, D), jnp.float32),
            pltpu.VMEM((pos_per_w, D), jnp.float32),
            pltpu.SemaphoreType.DMA,
            pltpu.SemaphoreType.DMA,
        ],
    )
    def k(idx_hbm, cnt_hbm, emb_hbm, out_hbm, idx_v, cnt_v, rows_v, out_v,
          sem0, sem1):
        wid = lax.axis_index("s") * _NC + lax.axis_index("c")
        sems = (sem0, sem1)
        pltpu.sync_copy(idx_hbm.at[wid], idx_v)
        pltpu.sync_copy(cnt_hbm.at[wid], cnt_v)
        n = cnt_v[...][0]

        def start(c, buf):
            pltpu.make_async_copy(
                emb_hbm.at[idx_v.at[c]], rows_v.at[buf], sems[buf]
            ).start()

        def wait(buf):
            pltpu.make_async_copy(
                emb_hbm.at[idx_v.at[0]], rows_v.at[buf], sems[buf]
            ).wait()

        def compute(c, buf):
            for p in range(chunk_pos):
                for d in range(D // 16):
                    sl = pl.ds(d * 16, 16)
                    acc = rows_v[buf, p * L, sl]
                    for l in range(1, L):
                        acc = acc + rows_v[buf, p * L + l, sl]
                    out_v[c * chunk_pos + p, sl] = acc

        @pl.when(n > 0)
        def _():
            start(0, 0)

        def body(i, carry):
            c = i * 2

            @pl.when(c + 1 < n)
            def _():
                start(c + 1, 1)

            wait(0)
            compute(c, 0)

            @pl.when(c + 2 < n)
            def _():
                start(c + 2, 0)

            @pl.when(c + 1 < n)
            def _():
                wait(1)
                compute(c + 1, 1)

            return carry

        lax.fori_loop(0, (n + 1) // 2, body, 0, unroll=False)
        pltpu.sync_copy(out_v, out_hbm.at[wid])

    return k(idx3, nchunks2, emb)


def _tc_entity_scan(tmax, encoded, maskf, keys, keysT, U, V, W):
    B, S, D = encoded.shape
    Kn = keys.shape[1]

    def body(tmax_ref, enc_ref, m_ref, keys_ref, keysT_ref, U_ref, V_ref,
             W_ref, out_ref, sw_ref, sk_ref, kv_ref):
        kv_ref[...] = jnp.dot(
            keys_ref[...].reshape(B * Kn, D), V_ref[...],
            preferred_element_type=jnp.float32).reshape(B, Kn, D)
        for b in range(B):
            enc_b = enc_ref[b]
            sw_ref[b] = jnp.dot(enc_b, W_ref[...],
                                preferred_element_type=jnp.float32)
            sk_ref[b] = jnp.dot(enc_b, keysT_ref[b],
                                preferred_element_type=jnp.float32)
        out_ref[...] = jnp.zeros((B, Kn, D), jnp.float32)
        kv = kv_ref[...]

        def step(t):
            h = out_ref[...]
            se = enc_ref[:, pl.ds(t, 1), :]
            hs = jnp.sum(h * se, axis=2)
            skt = sk_ref[:, pl.ds(t, 1), :].reshape(B, Kn)
            g = jax.nn.sigmoid(hs + skt)
            hU = jnp.dot(h.reshape(B * Kn, D), U_ref[...],
                         preferred_element_type=jnp.float32).reshape(B, Kn, D)
            swt = sw_ref[:, pl.ds(t, 1), :]
            ht = jnp.maximum(hU + kv + swt, 0.0)
            u = h + g[:, :, None] * ht
            nrm = lax.rsqrt(jnp.sum(u * u, axis=2, keepdims=True) + 1e-12)
            m = m_ref[:, pl.ds(t, 1), :]
            out_ref[...] = jnp.where(m > 0.5, u * nrm, h)

        def step4(i, carry):
            for j in range(4):
                step(i * 4 + j)
            return carry

        lax.fori_loop(0, (tmax_ref[0] + 3) // 4, step4, 0, unroll=False)

    return pl.pallas_call(
        body,
        out_shape=jax.ShapeDtypeStruct((B, Kn, D), jnp.float32),
        in_specs=[
            pl.BlockSpec(memory_space=pltpu.SMEM),
            pl.BlockSpec(memory_space=pltpu.VMEM),
            pl.BlockSpec(memory_space=pltpu.VMEM),
            pl.BlockSpec(memory_space=pltpu.VMEM),
            pl.BlockSpec(memory_space=pltpu.VMEM),
            pl.BlockSpec(memory_space=pltpu.VMEM),
            pl.BlockSpec(memory_space=pltpu.VMEM),
            pl.BlockSpec(memory_space=pltpu.VMEM),
        ],
        scratch_shapes=[
            pltpu.VMEM((B, S, D), jnp.float32),
            pltpu.VMEM((B, S, Kn), jnp.float32),
            pltpu.VMEM((B, Kn, D), jnp.float32),
        ],
    )(tmax, encoded, maskf[:, :, None], keys, keysT, U, V, W)


def kernel(paragraph, mask, keys, emb, U, V, W):
    B, S, L = paragraph.shape
    D = emb.shape[1]
    chunk_pos = 6
    w_per_b = _NW // B

    order = jnp.argsort(jnp.logical_not(mask), axis=1, stable=True)
    n_b = jnp.sum(mask, axis=1, dtype=jnp.int32)
    tmax = jnp.max(n_b).reshape(1)
    par_c = jnp.take_along_axis(paragraph.astype(jnp.int32),
                                order[:, :, None], axis=1)
    n_chunks_b = -(-S // chunk_pos)
    n_chunks_b += (-n_chunks_b) % w_per_b
    s_pad = n_chunks_b * chunk_pos
    half = n_chunks_b // w_per_b
    totc = (n_b + chunk_pos - 1) // chunk_pos
    h_arr = jnp.arange(_NW, dtype=jnp.int32) % w_per_b
    nchunks = (jnp.repeat(totc, w_per_b) - h_arr + (w_per_b - 1)) // w_per_b
    nchunks2 = jnp.broadcast_to(nchunks[:, None], (_NW, 16))

    par_pad = jnp.pad(par_c, ((0, 0), (0, s_pad - S), (0, 0)))
    pc = par_pad.reshape(B, half, w_per_b, chunk_pos * L)
    idx3 = jnp.swapaxes(pc, 1, 2).reshape(_NW, half, chunk_pos * L)
    out = _sc_gather_sum(idx3, nchunks2, emb, L, D, chunk_pos)
    enc = out.reshape(B, w_per_b, half, chunk_pos, D)
    encoded = jnp.swapaxes(enc, 1, 2).reshape(B, s_pad, D)[:, :S]

    vmask = (jnp.arange(S, dtype=jnp.int32)[None, :] < n_b[:, None])
    maskf = vmask.astype(jnp.float32)
    keysT = jnp.swapaxes(keys, 1, 2)
    return _tc_entity_scan(tmax, encoded, maskf, keys, keysT, U, V, W)


# speedup vs baseline: 1.0329x; 1.0329x over previous
import functools

import jax
import jax.numpy as jnp
from jax import lax
from jax.experimental import pallas as pl
from jax.experimental.pallas import tpu as pltpu
from jax.experimental.pallas import tpu_sc as plsc

_NC = 2
_NS = 16
_NW = _NC * _NS


def _sc_gather_sum(idx3, nchunks2, emb, L, D, chunk_pos):
    n_chunks = idx3.shape[1]
    pos_per_w = n_chunks * chunk_pos
    rows_per_chunk = chunk_pos * L
    mesh = plsc.VectorSubcoreMesh(core_axis_name="c", subcore_axis_name="s")

    @functools.partial(
        pl.kernel,
        out_type=jax.ShapeDtypeStruct((_NW, pos_per_w, D), jnp.float32),
        mesh=mesh,
        scratch_types=[
            pltpu.VMEM((n_chunks, rows_per_chunk), jnp.int32),
            pltpu.VMEM((16,), jnp.int32),
            pltpu.VMEM((2, rows_per_chunk, D), jnp.float32),
            pltpu.VMEM((pos_per_w, D), jnp.float32),
            pltpu.SemaphoreType.DMA,
            pltpu.SemaphoreType.DMA,
        ],
    )
    def k(idx_hbm, cnt_hbm, emb_hbm, out_hbm, idx_v, cnt_v, rows_v, out_v,
          sem0, sem1):
        wid = lax.axis_index("s") * _NC + lax.axis_index("c")
        sems = (sem0, sem1)
        pltpu.sync_copy(idx_hbm.at[wid], idx_v)
        pltpu.sync_copy(cnt_hbm.at[wid], cnt_v)
        n = cnt_v[...][0]

        def start(c, buf):
            pltpu.make_async_copy(
                emb_hbm.at[idx_v.at[c]], rows_v.at[buf], sems[buf]
            ).start()

        def wait(buf):
            pltpu.make_async_copy(
                emb_hbm.at[idx_v.at[0]], rows_v.at[buf], sems[buf]
            ).wait()

        def compute(c, buf):
            for p in range(chunk_pos):
                for d in range(D // 16):
                    sl = pl.ds(d * 16, 16)
                    acc = rows_v[buf, p * L, sl]
                    for l in range(1, L):
                        acc = acc + rows_v[buf, p * L + l, sl]
                    out_v[c * chunk_pos + p, sl] = acc

        @pl.when(n > 0)
        def _():
            start(0, 0)

        def body(i, carry):
            c = i * 2

            @pl.when(c + 1 < n)
            def _():
                start(c + 1, 1)

            wait(0)
            compute(c, 0)

            @pl.when(c + 2 < n)
            def _():
                start(c + 2, 0)

            @pl.when(c + 1 < n)
            def _():
                wait(1)
                compute(c + 1, 1)

            return carry

        lax.fori_loop(0, (n + 1) // 2, body, 0, unroll=False)
        pltpu.sync_copy(out_v, out_hbm.at[wid])

    return k(idx3, nchunks2, emb)


def _tc_entity_scan(tmax, encoded, maskf, keys, keysT, U, V, W):
    B, S, D = encoded.shape
    Kn = keys.shape[1]

    def body(tmax_ref, enc_ref, m_ref, keys_ref, keysT_ref, U_ref, V_ref,
             W_ref, out_ref, sw_ref, sk_ref, kv_ref):
        kv_ref[...] = jnp.dot(
            keys_ref[...].reshape(B * Kn, D), V_ref[...],
            preferred_element_type=jnp.float32).reshape(B, Kn, D)
        for b in range(B):
            enc_b = enc_ref[b]
            sw_ref[b] = jnp.dot(enc_b, W_ref[...],
                                preferred_element_type=jnp.float32)
            sk_ref[b] = jnp.dot(enc_b, keysT_ref[b],
                                preferred_element_type=jnp.float32)
        out_ref[...] = jnp.zeros((B, Kn, D), jnp.float32)
        kv = kv_ref[...]

        def step(t):
            h = out_ref[...]
            se = enc_ref[:, pl.ds(t, 1), :]
            hs = jnp.sum(h * se, axis=2)
            skt = sk_ref[:, pl.ds(t, 1), :].reshape(B, Kn)
            g = jax.nn.sigmoid(hs + skt)
            hU = jnp.dot(h.reshape(B * Kn, D), U_ref[...],
                         preferred_element_type=jnp.float32).reshape(B, Kn, D)
            swt = sw_ref[:, pl.ds(t, 1), :]
            ht = jnp.maximum(hU + kv + swt, 0.0)
            u = h + g[:, :, None] * ht
            nrm = lax.rsqrt(jnp.sum(u * u, axis=2, keepdims=True) + 1e-12)
            m = m_ref[:, pl.ds(t, 1), :]
            out_ref[...] = jnp.where(m > 0.5, u * nrm, h)

        def step4(i, carry):
            for j in range(4):
                step(i * 4 + j)
            return carry

        lax.fori_loop(0, (tmax_ref[0] + 3) // 4, step4, 0, unroll=False)

    return pl.pallas_call(
        body,
        out_shape=jax.ShapeDtypeStruct((B, Kn, D), jnp.float32),
        in_specs=[
            pl.BlockSpec(memory_space=pltpu.SMEM),
            pl.BlockSpec(memory_space=pltpu.VMEM),
            pl.BlockSpec(memory_space=pltpu.VMEM),
            pl.BlockSpec(memory_space=pltpu.VMEM),
            pl.BlockSpec(memory_space=pltpu.VMEM),
            pl.BlockSpec(memory_space=pltpu.VMEM),
            pl.BlockSpec(memory_space=pltpu.VMEM),
            pl.BlockSpec(memory_space=pltpu.VMEM),
        ],
        scratch_shapes=[
            pltpu.VMEM((B, S, D), jnp.float32),
            pltpu.VMEM((B, S, Kn), jnp.float32),
            pltpu.VMEM((B, Kn, D), jnp.float32),
        ],
    )(tmax, encoded, maskf[:, :, None], keys, keysT, U, V, W)


def kernel(paragraph, mask, keys, emb, U, V, W):
    B, S, L = paragraph.shape
    D = emb.shape[1]
    chunk_pos = 4
    w_per_b = _NW // B

    order = jnp.argsort(jnp.logical_not(mask), axis=1, stable=True)
    n_b = jnp.sum(mask, axis=1, dtype=jnp.int32)
    tmax = jnp.max(n_b).reshape(1)
    par_c = jnp.take_along_axis(paragraph.astype(jnp.int32),
                                order[:, :, None], axis=1)
    n_chunks_b = -(-S // chunk_pos)
    n_chunks_b += (-n_chunks_b) % w_per_b
    s_pad = n_chunks_b * chunk_pos
    half = n_chunks_b // w_per_b
    totc = (n_b + chunk_pos - 1) // chunk_pos
    h_arr = jnp.arange(_NW, dtype=jnp.int32) % w_per_b
    nchunks = (jnp.repeat(totc, w_per_b) - h_arr + (w_per_b - 1)) // w_per_b
    nchunks2 = jnp.broadcast_to(nchunks[:, None], (_NW, 16))

    par_pad = jnp.pad(par_c, ((0, 0), (0, s_pad - S), (0, 0)))
    pc = par_pad.reshape(B, half, w_per_b, chunk_pos * L)
    idx3 = jnp.swapaxes(pc, 1, 2).reshape(_NW, half, chunk_pos * L)
    out = _sc_gather_sum(idx3, nchunks2, emb, L, D, chunk_pos)
    enc = out.reshape(B, w_per_b, half, chunk_pos, D)
    encoded = jnp.swapaxes(enc, 1, 2).reshape(B, s_pad, D)[:, :S]

    vmask = (jnp.arange(S, dtype=jnp.int32)[None, :] < n_b[:, None])
    maskf = vmask.astype(jnp.float32)
    keysT = jnp.swapaxes(keys, 1, 2)
    return _tc_entity_scan(tmax, encoded, maskf, keys, keysT, U, V, W)
